# baseline (speedup 1.0000x reference)
import functools

import jax
import jax.numpy as jnp
from jax import lax
from jax.experimental import pallas as pl
from jax.experimental.pallas import tpu as pltpu
from jax.experimental.pallas import tpu_sc as plsc

B, S, D = 2, 2048, 1024
T = 16
L = 10
LANES = 16
NC, NS = 2, 16
NW = NC * NS
ROWS = B * S
RPW = ROWS // NW
WPB = NW // B
G = 2
CH = D // LANES

_mesh = plsc.VectorSubcoreMesh(core_axis_name="c", subcore_axis_name="s")


@functools.partial(
    pl.kernel,
    mesh=_mesh,
    out_type=jax.ShapeDtypeStruct((B, T, S, D), jnp.float32),
    scratch_types=[
        pltpu.VMEM((LANES,), jnp.float32),
        pltpu.VMEM((G * D,), jnp.float32),
        pltpu.VMEM((T * G * D,), jnp.float32),
        pltpu.VMEM((G * D,), jnp.int32),
        pltpu.SemaphoreType.DMA,
    ],
    compiler_params=pltpu.CompilerParams(needs_layout_passes=False),
)
def _sc_spikes(x_hbm, scal_hbm, out_hbm, scal_v, xrow_v, tile_v, stbuf_v, sem_out):
    wid = lax.axis_index("s") * NC + lax.axis_index("c")
    b = wid // WPB
    s_base = (wid % WPB) * RPW

    pltpu.sync_copy(scal_hbm, scal_v)
    nscal = -scal_v[...]
    ones = jnp.full((LANES,), 1.0, jnp.float32)
    zeros = jnp.zeros((LANES,), jnp.float32)
    col0 = lax.iota(jnp.int32, LANES)

    def _zchunk(c, carry):
        tile_v[pl.ds(c * LANES, LANES)] = zeros
        return carry

    lax.fori_loop(0, T * G * CH, _zchunk, 0)

    def _group(i, carry):
        s = s_base + i * G
        for g in range(G):
            pltpu.sync_copy(x_hbm.at[b, s + g], xrow_v.at[pl.ds(g * D, D)])
        for g in range(G):
            for c in range(CH):
                xv = xrow_v[pl.ds(g * D + c * LANES, LANES)]
                e = jnp.exp(xv * nscal)
                v = (e * (L + 0.5) + 0.5) / (e + 1.0)
                idx = (v.astype(jnp.int32) * G + g) * D + (col0 + c * LANES)
                plsc.store_scatter(tile_v, [idx], ones)
                stbuf_v[pl.ds((g * CH + c) * LANES, LANES)] = idx
        copies = []
        for t in range(T):
            for g in range(G):
                copies.append(
                    pltpu.async_copy(
                        tile_v.at[pl.ds((t * G + g) * D, D)],
                        out_hbm.at[b, t, s + g],
                        sem_out,
                    )
                )
        for cp in copies:
            cp.wait()
        for g in range(G):
            for c in range(CH):
                idx = stbuf_v[pl.ds((g * CH + c) * LANES, LANES)]
                plsc.store_scatter(tile_v, [idx], zeros)
        return carry

    lax.fori_loop(0, RPW // G, _group, 0)


def kernel(x, scaling):
    scal16 = jnp.broadcast_to(scaling, (LANES,)).astype(jnp.float32)
    return _sc_spikes(x, scal16)


# speedup vs baseline: 1.2735x; 1.2735x over previous
import functools

import jax
import jax.numpy as jnp
from jax import lax
from jax.experimental import pallas as pl
from jax.experimental.pallas import tpu as pltpu
from jax.experimental.pallas import tpu_sc as plsc

B, S, D = 2, 2048, 1024
T = 16
L = 10
LANES = 16
NC, NS = 2, 16
NW = NC * NS
ROWS = B * S
RPW = ROWS // NW
WPB = NW // B
G = 2
CH = D // LANES
NG = RPW // G
CPG = G * CH

_mesh = plsc.VectorSubcoreMesh(core_axis_name="c", subcore_axis_name="s")


@functools.partial(
    pl.kernel,
    mesh=_mesh,
    out_type=jax.ShapeDtypeStruct((B, T, S, D), jnp.float32),
    scratch_types=[
        pltpu.VMEM((LANES,), jnp.float32),
        pltpu.VMEM((G * D,), jnp.float32),
        pltpu.VMEM((G * D,), jnp.float32),
        pltpu.VMEM((T * G * D,), jnp.float32),
        pltpu.VMEM((T * G * D,), jnp.float32),
        pltpu.VMEM((G * D,), jnp.int32),
        pltpu.VMEM((G * D,), jnp.int32),
        pltpu.SemaphoreType.DMA,
        pltpu.SemaphoreType.DMA,
    ],
    compiler_params=pltpu.CompilerParams(needs_layout_passes=False),
)
def _sc_spikes(x_hbm, scal_hbm, out_hbm, scal_v, xr0, xr1, tl0, tl1, sb0, sb1,
               sem0, sem1):
    wid = lax.axis_index("s") * NC + lax.axis_index("c")
    b = wid // WPB
    s_base = (wid % WPB) * RPW

    xrow = (xr0, xr1)
    tile = (tl0, tl1)
    stbuf = (sb0, sb1)
    sems = (sem0, sem1)

    pltpu.sync_copy(scal_hbm, scal_v)
    nscal = -scal_v[...]
    ones = jnp.full((LANES,), 1.0, jnp.float32)
    zeros = jnp.zeros((LANES,), jnp.float32)
    col0 = lax.iota(jnp.int32, LANES)

    def _zchunk(c, carry):
        tl0[pl.ds(c * LANES, LANES)] = zeros
        tl1[pl.ds(c * LANES, LANES)] = zeros
        return carry

    lax.fori_loop(0, T * G * CH, _zchunk, 0)

    def _compute(k, s):
        for g in range(G):
            pltpu.sync_copy(x_hbm.at[b, s + g], xrow[k].at[pl.ds(g * D, D)])

        def _chunk(j, carry):
            g = j // CH
            d0 = (j % CH) * LANES
            xv = xrow[k][pl.ds(j * LANES, LANES)]
            e = jnp.exp(xv * nscal)
            v = (e * (L + 0.5) + 0.5) / (e + 1.0)
            idx = (v.astype(jnp.int32) * G + g) * D + (col0 + d0)
            plsc.store_scatter(tile[k], [idx], ones)
            stbuf[k][pl.ds(j * LANES, LANES)] = idx
            return carry

        lax.fori_loop(0, CPG, _chunk, 0)
        for t in range(T):
            for g in range(G):
                pltpu.async_copy(
                    tile[k].at[pl.ds((t * G + g) * D, D)],
                    out_hbm.at[b, t, s + g],
                    sems[k],
                )

    def _drain(k):
        for _ in range(T * G):
            pltpu.make_async_copy(
                tile[k].at[pl.ds(0, D)], out_hbm.at[b, 0, s_base], sems[k]
            ).wait()

    def _rezero(k):
        def _zc(j, carry):
            idx = stbuf[k][pl.ds(j * LANES, LANES)]
            plsc.store_scatter(tile[k], [idx], zeros)
            return carry

        lax.fori_loop(0, CPG, _zc, 0)

    _compute(0, s_base)
    _compute(1, s_base + G)

    def _pair(i, carry):
        s = s_base + (2 * i + 2) * G
        for k in range(2):
            _drain(k)
            _rezero(k)
            _compute(k, s + k * G)
        return carry

    lax.fori_loop(0, NG // 2 - 1, _pair, 0)
    _drain(0)
    _drain(1)


def kernel(x, scaling):
    scal16 = jnp.broadcast_to(scaling, (LANES,)).astype(jnp.float32)
    return _sc_spikes(x, scal16)


# speedup vs baseline: 1.3946x; 1.0951x over previous
import functools
import math

import jax
import jax.numpy as jnp
from jax import lax
from jax.experimental import pallas as pl
from jax.experimental.pallas import tpu as pltpu
from jax.experimental.pallas import tpu_sc as plsc

B, S, D = 2, 2048, 1024
T = 16
L = 10
LANES = 16
NC, NS = 2, 16
NW = NC * NS
ROWS = B * S
RPW = ROWS // NW
WPB = NW // B
G = 2
CH = D // LANES
NG = RPW // G
UNROLL = 4

_TH = [math.log((L - k - 0.5) / (k + 0.5)) for k in range(L)]

_mesh = plsc.VectorSubcoreMesh(core_axis_name="c", subcore_axis_name="s")


@functools.partial(
    pl.kernel,
    mesh=_mesh,
    out_type=jax.ShapeDtypeStruct((B, T, S, D), jnp.float32),
    scratch_types=[
        pltpu.VMEM((LANES,), jnp.float32),
        pltpu.VMEM((G * D,), jnp.float32),
        pltpu.VMEM((G * D,), jnp.float32),
        pltpu.VMEM((G * T, D), jnp.float32),
        pltpu.VMEM((G * T, D), jnp.float32),
        pltpu.VMEM((G * D,), jnp.int32),
        pltpu.VMEM((G * D,), jnp.int32),
        pltpu.SemaphoreType.DMA,
        pltpu.SemaphoreType.DMA,
    ],
    compiler_params=pltpu.CompilerParams(needs_layout_passes=False),
)
def _sc_spikes(x_hbm, scal_hbm, out_hbm, scal_v, xr0, xr1, tl0, tl1, sb0, sb1,
               sem0, sem1):
    wid = lax.axis_index("s") * NC + lax.axis_index("c")
    b = wid // WPB
    s_base = (wid % WPB) * RPW

    xrow = (xr0, xr1)
    tile = (tl0, tl1)
    stbuf = (sb0, sb1)
    sems = (sem0, sem1)

    pltpu.sync_copy(scal_hbm, scal_v)
    scal = scal_v[...]
    ones = jnp.full((LANES,), 1.0, jnp.float32)
    zeros = jnp.zeros((LANES,), jnp.float32)
    izeros = jnp.zeros((LANES,), jnp.int32)
    col0 = lax.iota(jnp.int32, LANES)

    def _zrow(r, carry):
        def _zc(c, carry2):
            tile[0][r, pl.ds(c * LANES, LANES)] = zeros
            tile[1][r, pl.ds(c * LANES, LANES)] = zeros
            return carry2
        return lax.fori_loop(0, CH, _zc, carry)

    lax.fori_loop(0, G * T, _zrow, 0)

    def _zst(c, carry):
        stbuf[0][pl.ds(c * LANES, LANES)] = izeros
        stbuf[1][pl.ds(c * LANES, LANES)] = izeros
        return carry

    lax.fori_loop(0, G * CH, _zst, 0)

    def _process(k, s):
        for g in range(G):
            pltpu.sync_copy(x_hbm.at[b, s + g], xrow[k].at[pl.ds(g * D, D)])
        for g in range(G):
            def _chunk(j, carry):
                for u in range(UNROLL):
                    off = g * D + (j * UNROLL + u) * LANES
                    dvec = col0 + ((j * UNROLL + u) * LANES)
                    old = stbuf[k][pl.ds(off, LANES)]
                    plsc.store_scatter(tile[k], [old, dvec], zeros)
                    z = xrow[k][pl.ds(off, LANES)] * scal
                    st = jnp.where(z <= _TH[0], 1, 0)
                    for th in _TH[1:]:
                        st = st + jnp.where(z <= th, 1, 0)
                    row = st + (g * T)
                    plsc.store_scatter(tile[k], [row, dvec], ones)
                    stbuf[k][pl.ds(off, LANES)] = row
                return carry

            lax.fori_loop(0, CH // UNROLL, _chunk, 0)
        for g in range(G):
            pltpu.async_copy(
                tile[k].at[pl.ds(g * T, T)],
                out_hbm.at[b, pl.ds(0, T), s + g],
                sems[k],
            )

    def _drain(k):
        for _ in range(G):
            pltpu.make_async_copy(
                tile[k].at[pl.ds(0, T)],
                out_hbm.at[b, pl.ds(0, T), s_base],
                sems[k],
            ).wait()

    _process(0, s_base)
    _process(1, s_base + G)

    def _pair(i, carry):
        s = s_base + (2 * i + 2) * G
        for k in range(2):
            _drain(k)
            _process(k, s + k * G)
        return carry

    lax.fori_loop(0, NG // 2 - 1, _pair, 0)
    _drain(0)
    _drain(1)


def kernel(x, scaling):
    scal16 = jnp.broadcast_to(scaling, (LANES,)).astype(jnp.float32)
    return _sc_spikes(x, scal16)
